# baseline (speedup 1.0000x reference)
import functools

import jax
import jax.numpy as jnp
from jax import lax
from jax.experimental import pallas as pl
from jax.experimental.pallas import tpu as pltpu
from jax.experimental.pallas import tpu_sc as plsc

N_NODES = 10000
N_EDGES = 320000
D_EDGE = 16
D_IN = 128
D_OUT = 128

NC = 2
NS = 16
NW = NC * NS

LANES = 16
EROWS = N_EDGES // 128
ROWS_BASE = EROWS // NW
ROWS_REM = EROWS % NW
NPAD = 10240


def _sc_body(et_hbm, tgt_hbm, sums_hbm, counts_hbm, idx_v, rows_v, ones_v,
             zc_v, cp_v, tv_v, st_v, acc_sh, cnt_sh):
    c = lax.axis_index("c")
    s = lax.axis_index("s")
    wid = c * NS + s

    zero16 = jnp.zeros((LANES,), jnp.float32)
    ones16 = jnp.ones((LANES,), jnp.float32)

    def zrow(i, _):
        rows_v[i] = zero16
        return 0
    lax.fori_loop(0, 128, zrow, 0)
    for k in range(8):
        ones_v[pl.ds(k * LANES, LANES)] = ones16

    def zc(i, _):
        zc_v[pl.ds(i * LANES, LANES)] = zero16
        return 0
    lax.fori_loop(0, 40, zc, 0)

    for k in range(5):
        pltpu.sync_copy(rows_v, acc_sh.at[pl.ds(s * 640 + k * 128, 128)])
    pltpu.sync_copy(zc_v, cnt_sh.at[pl.ds(s * 640, 640)])

    plsc.subcore_barrier()

    start = ROWS_BASE * wid + jnp.minimum(wid, ROWS_REM)
    cnt = ROWS_BASE + jnp.where(wid < ROWS_REM, 1, 0)

    iota16c = lax.iota(jnp.int32, LANES)
    zeros16i = jnp.zeros((LANES,), jnp.int32)

    def body(r, _):
        pltpu.sync_copy(tgt_hbm.at[r], idx_v)
        pltpu.sync_copy(et_hbm.at[:, pl.ds(r * 128, 128)], st_v)

        def tpose(j, _):
            for eo in range(LANES):
                e = j * LANES + eo
                vals = plsc.load_gather(st_v, [iota16c, zeros16i + e])
                rows_v[e] = vals
            return 0
        lax.fori_loop(0, 128 // LANES, tpose, 0)

        pltpu.sync_copy(rows_v, acc_sh.at[idx_v], add=True)
        pltpu.sync_copy(ones_v, cnt_sh.at[idx_v], add=True)
        return 0

    lax.fori_loop(start, start + cnt, body, 0)

    plsc.subcore_barrier()

    pltpu.sync_copy(acc_sh.at[pl.ds(s * 640, 640)], cp_v)
    iota16 = lax.iota(jnp.int32, LANES)

    def tbody(j, _):
        rows = j * LANES + iota16
        for d in range(D_EDGE):
            vals = plsc.load_gather(
                cp_v, [rows, jnp.full((LANES,), d, jnp.int32)])
            tv_v[d, pl.ds(j * LANES, LANES)] = vals
        return 0

    lax.fori_loop(0, 640 // LANES, tbody, 0)
    pltpu.sync_copy(tv_v, sums_hbm.at[c, :, pl.ds(s * 640, 640)])
    pltpu.sync_copy(cnt_sh.at[pl.ds(s * 640, 640)],
                    counts_hbm.at[c, pl.ds(s * 640, 640)])


def _sc_segment_sum(edge_attr_t, targets2d):
    mesh = plsc.VectorSubcoreMesh(
        core_axis_name="c", subcore_axis_name="s", num_cores=NC,
        num_subcores=NS)
    f = functools.partial(
        pl.kernel,
        out_type=[
            jax.ShapeDtypeStruct((NC, D_EDGE, NPAD), jnp.float32),
            jax.ShapeDtypeStruct((NC, NPAD), jnp.float32),
        ],
        mesh=mesh,
        compiler_params=pltpu.CompilerParams(
            needs_layout_passes=False, use_tc_tiling_on_sc=False),
        scratch_types=[
            pltpu.VMEM((128,), jnp.int32),
            pltpu.VMEM((128, D_EDGE), jnp.float32),
            pltpu.VMEM((128,), jnp.float32),
            pltpu.VMEM((640,), jnp.float32),
            pltpu.VMEM((640, D_EDGE), jnp.float32),
            pltpu.VMEM((D_EDGE, 640), jnp.float32),
            pltpu.VMEM((D_EDGE, 128), jnp.float32),
            pltpu.VMEM_SHARED((NPAD, D_EDGE), jnp.float32),
            pltpu.VMEM_SHARED((NPAD,), jnp.float32),
        ],
    )(_sc_body)
    return f(edge_attr_t, targets2d)


def _tc_body(node_ref, sums_ref, counts_ref, wn_ref, we_ref, b_ref, out_ref):
    s_t = sums_ref[0] + sums_ref[1]
    cnts = counts_ref[0] + counts_ref[1]
    mean_t = s_t / jnp.maximum(cnts, 1.0)[None, :]
    acc = jnp.dot(node_ref[...], wn_ref[...], preferred_element_type=jnp.float32)
    acc += lax.dot_general(mean_t, we_ref[...], (((0,), (0,)), ((), ())),
                           preferred_element_type=jnp.float32)
    out_ref[...] = jax.nn.sigmoid(acc + b_ref[...])


def _tc_finish(node_attr, sums, counts, wn, we, b2d):
    blk = 1024
    grid = pl.cdiv(N_NODES, blk)
    return pl.pallas_call(
        _tc_body,
        grid=(grid,),
        in_specs=[
            pl.BlockSpec((blk, D_IN), lambda i: (i, 0)),
            pl.BlockSpec((NC, D_EDGE, blk), lambda i: (0, 0, i)),
            pl.BlockSpec((NC, blk), lambda i: (0, i)),
            pl.BlockSpec((D_IN, D_OUT), lambda i: (0, 0)),
            pl.BlockSpec((D_EDGE, D_OUT), lambda i: (0, 0)),
            pl.BlockSpec((1, D_OUT), lambda i: (0, 0)),
        ],
        out_specs=pl.BlockSpec((blk, D_OUT), lambda i: (i, 0)),
        out_shape=jax.ShapeDtypeStruct((N_NODES, D_OUT), jnp.float32),
    )(node_attr, sums, counts, wn, we, b2d)


@jax.jit
def kernel(edge_attr, edge_index, node_attr, W, b):
    targets2d = edge_index[0].reshape(EROWS, 128)
    sums, counts = _sc_segment_sum(edge_attr.T, targets2d)
    wn = W[:, :D_IN].T
    we = W[:, D_IN:].T
    return _tc_finish(node_attr, sums, counts, wn, we, b.reshape(1, D_OUT))


# speedup vs baseline: 1.4333x; 1.4333x over previous
import functools

import jax
import jax.numpy as jnp
from jax import lax
from jax.experimental import pallas as pl
from jax.experimental.pallas import tpu as pltpu
from jax.experimental.pallas import tpu_sc as plsc

N_NODES = 10000
N_EDGES = 320000
D_EDGE = 16
D_IN = 128
D_OUT = 128

NC = 2
NS = 16
NW = NC * NS

LANES = 16
EROWS = N_EDGES // 128
ROWS_BASE = EROWS // NW
ROWS_REM = EROWS % NW
NPAD = 10240


def _sc_body(et_hbm, tgt_hbm, sums_hbm, counts_hbm, idx_v, ones_v, zc_v,
             st_v, acc2_sh, cnt_sh, sem):
    c = lax.axis_index("c")
    s = lax.axis_index("s")
    wid = c * NS + s

    zero16 = jnp.zeros((LANES,), jnp.float32)
    ones16 = jnp.ones((LANES,), jnp.float32)

    for k in range(8):
        ones_v[pl.ds(k * LANES, LANES)] = ones16

    def zc(i, _):
        zc_v[pl.ds(i * LANES, LANES)] = zero16
        return 0
    lax.fori_loop(0, 40, zc, 0)

    for d in range(D_EDGE):
        pltpu.sync_copy(zc_v, acc2_sh.at[d, pl.ds(s * 640, 640)])
    pltpu.sync_copy(zc_v, cnt_sh.at[pl.ds(s * 640, 640)])

    plsc.subcore_barrier()

    start = ROWS_BASE * wid + jnp.minimum(wid, ROWS_REM)
    cnt = ROWS_BASE + jnp.where(wid < ROWS_REM, 1, 0)

    def body(r, _):
        pltpu.sync_copy(tgt_hbm.at[r], idx_v)
        pltpu.sync_copy(et_hbm.at[:, pl.ds(r * 128, 128)], st_v)
        hs = [
            pltpu.async_copy(st_v.at[d], acc2_sh.at[d].at[idx_v], sem,
                             add=True)
            for d in range(D_EDGE)
        ]
        hc = pltpu.async_copy(ones_v, cnt_sh.at[idx_v], sem, add=True)
        for h in hs:
            h.wait()
        hc.wait()
        return 0

    lax.fori_loop(start, start + cnt, body, 0)

    plsc.subcore_barrier()

    pltpu.sync_copy(acc2_sh.at[:, pl.ds(s * 640, 640)],
                    sums_hbm.at[c, :, pl.ds(s * 640, 640)])
    pltpu.sync_copy(cnt_sh.at[pl.ds(s * 640, 640)],
                    counts_hbm.at[c, pl.ds(s * 640, 640)])


def _sc_segment_sum(edge_attr_t, targets2d):
    mesh = plsc.VectorSubcoreMesh(
        core_axis_name="c", subcore_axis_name="s", num_cores=NC,
        num_subcores=NS)
    f = functools.partial(
        pl.kernel,
        out_type=[
            jax.ShapeDtypeStruct((NC, D_EDGE, NPAD), jnp.float32),
            jax.ShapeDtypeStruct((NC, NPAD), jnp.float32),
        ],
        mesh=mesh,
        compiler_params=pltpu.CompilerParams(
            needs_layout_passes=False, use_tc_tiling_on_sc=False),
        scratch_types=[
            pltpu.VMEM((128,), jnp.int32),
            pltpu.VMEM((128,), jnp.float32),
            pltpu.VMEM((640,), jnp.float32),
            pltpu.VMEM((D_EDGE, 128), jnp.float32),
            pltpu.VMEM_SHARED((D_EDGE, NPAD), jnp.float32),
            pltpu.VMEM_SHARED((NPAD,), jnp.float32),
            pltpu.SemaphoreType.DMA,
        ],
    )(_sc_body)
    return f(edge_attr_t, targets2d)


def _tc_body(node_ref, sums_ref, counts_ref, wn_ref, we_ref, b_ref, out_ref):
    s_t = sums_ref[0] + sums_ref[1]
    cnts = counts_ref[0] + counts_ref[1]
    mean_t = s_t / jnp.maximum(cnts, 1.0)[None, :]
    acc = jnp.dot(node_ref[...], wn_ref[...], preferred_element_type=jnp.float32)
    acc += lax.dot_general(mean_t, we_ref[...], (((0,), (0,)), ((), ())),
                           preferred_element_type=jnp.float32)
    out_ref[...] = jax.nn.sigmoid(acc + b_ref[...])


def _tc_finish(node_attr, sums, counts, wn, we, b2d):
    blk = 1024
    grid = pl.cdiv(N_NODES, blk)
    return pl.pallas_call(
        _tc_body,
        grid=(grid,),
        in_specs=[
            pl.BlockSpec((blk, D_IN), lambda i: (i, 0)),
            pl.BlockSpec((NC, D_EDGE, blk), lambda i: (0, 0, i)),
            pl.BlockSpec((NC, blk), lambda i: (0, i)),
            pl.BlockSpec((D_IN, D_OUT), lambda i: (0, 0)),
            pl.BlockSpec((D_EDGE, D_OUT), lambda i: (0, 0)),
            pl.BlockSpec((1, D_OUT), lambda i: (0, 0)),
        ],
        out_specs=pl.BlockSpec((blk, D_OUT), lambda i: (i, 0)),
        out_shape=jax.ShapeDtypeStruct((N_NODES, D_OUT), jnp.float32),
    )(node_attr, sums, counts, wn, we, b2d)


@jax.jit
def kernel(edge_attr, edge_index, node_attr, W, b):
    targets2d = edge_index[0].reshape(EROWS, 128)
    sums, counts = _sc_segment_sum(edge_attr.T, targets2d)
    wn = W[:, :D_IN].T
    we = W[:, D_IN:].T
    return _tc_finish(node_attr, sums, counts, wn, we, b.reshape(1, D_OUT))


# speedup vs baseline: 2.4429x; 1.7044x over previous
import functools

import jax
import jax.numpy as jnp
from jax import lax
from jax.experimental import pallas as pl
from jax.experimental.pallas import tpu as pltpu
from jax.experimental.pallas import tpu_sc as plsc

N_NODES = 10000
N_EDGES = 320000
D_EDGE = 16
D_IN = 128
D_OUT = 128

NC = 2
NS = 16
NW = NC * NS

LANES = 16
EROWS = N_EDGES // 128
ROWS_BASE = EROWS // NW
ROWS_REM = EROWS % NW
NPAD = 10240


def _sc_body(et_hbm, tgt_hbm, sums_hbm, counts_hbm, idx_v, ones_v, zc_v,
             st_v, acc2_sh, cnt_sh, sem, sem_i, sem_s):
    c = lax.axis_index("c")
    s = lax.axis_index("s")
    wid = c * NS + s

    zero16 = jnp.zeros((LANES,), jnp.float32)
    ones16 = jnp.ones((LANES,), jnp.float32)

    for k in range(8):
        ones_v[pl.ds(k * LANES, LANES)] = ones16

    def zc(i, _):
        zc_v[pl.ds(i * LANES, LANES)] = zero16
        return 0
    lax.fori_loop(0, 40, zc, 0)

    for d in range(D_EDGE):
        pltpu.sync_copy(zc_v, acc2_sh.at[d, pl.ds(s * 640, 640)])
    pltpu.sync_copy(zc_v, cnt_sh.at[pl.ds(s * 640, 640)])

    plsc.subcore_barrier()

    start = ROWS_BASE * wid + jnp.minimum(wid, ROWS_REM)
    cnt = ROWS_BASE + jnp.where(wid < ROWS_REM, 1, 0)
    end = start + cnt

    def issue_loads(r, slot):
        pltpu.async_copy(tgt_hbm.at[r], idx_v.at[slot], sem_i)
        pltpu.async_copy(et_hbm.at[:, pl.ds(r * 128, 128)], st_v.at[slot],
                         sem_s)

    issue_loads(start, 0)

    def body(r, _):
        slot = lax.rem(r - start, 2)
        pltpu.make_async_copy(tgt_hbm.at[r], idx_v.at[slot], sem_i).wait()
        pltpu.make_async_copy(et_hbm.at[:, pl.ds(r * 128, 128)],
                              st_v.at[slot], sem_s).wait()

        @pl.when(r + 1 < end)
        def _():
            issue_loads(r + 1, 1 - slot)

        hs = [
            pltpu.async_copy(st_v.at[slot, d], acc2_sh.at[d].at[idx_v.at[slot]],
                             sem, add=True)
            for d in range(D_EDGE)
        ]
        hc = pltpu.async_copy(ones_v, cnt_sh.at[idx_v.at[slot]], sem, add=True)
        for h in hs:
            h.wait()
        hc.wait()
        return 0

    lax.fori_loop(start, end, body, 0)

    plsc.subcore_barrier()

    pltpu.sync_copy(acc2_sh.at[:, pl.ds(s * 640, 640)],
                    sums_hbm.at[c, :, pl.ds(s * 640, 640)])
    pltpu.sync_copy(cnt_sh.at[pl.ds(s * 640, 640)],
                    counts_hbm.at[c, pl.ds(s * 640, 640)])


def _sc_segment_sum(edge_attr_t, targets2d):
    mesh = plsc.VectorSubcoreMesh(
        core_axis_name="c", subcore_axis_name="s", num_cores=NC,
        num_subcores=NS)
    f = functools.partial(
        pl.kernel,
        out_type=[
            jax.ShapeDtypeStruct((NC, D_EDGE, NPAD), jnp.float32),
            jax.ShapeDtypeStruct((NC, NPAD), jnp.float32),
        ],
        mesh=mesh,
        compiler_params=pltpu.CompilerParams(
            needs_layout_passes=False, use_tc_tiling_on_sc=False),
        scratch_types=[
            pltpu.VMEM((2, 128), jnp.int32),
            pltpu.VMEM((128,), jnp.float32),
            pltpu.VMEM((640,), jnp.float32),
            pltpu.VMEM((2, D_EDGE, 128), jnp.float32),
            pltpu.VMEM_SHARED((D_EDGE, NPAD), jnp.float32),
            pltpu.VMEM_SHARED((NPAD,), jnp.float32),
            pltpu.SemaphoreType.DMA,
            pltpu.SemaphoreType.DMA,
            pltpu.SemaphoreType.DMA,
        ],
    )(_sc_body)
    return f(edge_attr_t, targets2d)


def _tc_body(node_ref, sums_ref, counts_ref, wn_ref, we_ref, b_ref, out_ref):
    s_t = sums_ref[0] + sums_ref[1]
    cnts = counts_ref[0] + counts_ref[1]
    mean_t = s_t / jnp.maximum(cnts, 1.0)[None, :]
    acc = jnp.dot(node_ref[...], wn_ref[...], preferred_element_type=jnp.float32)
    acc += lax.dot_general(mean_t, we_ref[...], (((0,), (0,)), ((), ())),
                           preferred_element_type=jnp.float32)
    out_ref[...] = jax.nn.sigmoid(acc + b_ref[...])


def _tc_finish(node_attr, sums, counts, wn, we, b2d):
    blk = 1024
    grid = pl.cdiv(N_NODES, blk)
    return pl.pallas_call(
        _tc_body,
        grid=(grid,),
        in_specs=[
            pl.BlockSpec((blk, D_IN), lambda i: (i, 0)),
            pl.BlockSpec((NC, D_EDGE, blk), lambda i: (0, 0, i)),
            pl.BlockSpec((NC, blk), lambda i: (0, i)),
            pl.BlockSpec((D_IN, D_OUT), lambda i: (0, 0)),
            pl.BlockSpec((D_EDGE, D_OUT), lambda i: (0, 0)),
            pl.BlockSpec((1, D_OUT), lambda i: (0, 0)),
        ],
        out_specs=pl.BlockSpec((blk, D_OUT), lambda i: (i, 0)),
        out_shape=jax.ShapeDtypeStruct((N_NODES, D_OUT), jnp.float32),
    )(node_attr, sums, counts, wn, we, b2d)


@jax.jit
def kernel(edge_attr, edge_index, node_attr, W, b):
    targets2d = edge_index[0].reshape(EROWS, 128)
    sums, counts = _sc_segment_sum(edge_attr.T, targets2d)
    wn = W[:, :D_IN].T
    we = W[:, D_IN:].T
    return _tc_finish(node_attr, sums, counts, wn, we, b.reshape(1, D_OUT))


# speedup vs baseline: 2.6628x; 1.0900x over previous
import functools

import jax
import jax.numpy as jnp
from jax import lax
from jax.experimental import pallas as pl
from jax.experimental.pallas import tpu as pltpu
from jax.experimental.pallas import tpu_sc as plsc

N_NODES = 10000
N_EDGES = 320000
D_EDGE = 16
D_IN = 128
D_OUT = 128

NC = 2
NS = 16
NW = NC * NS

LANES = 16
EROWS = N_EDGES // 128
ROWS_BASE = EROWS // NW
ROWS_REM = EROWS % NW
NPAD = 10240


def _sc_body(et_hbm, tgt_hbm, sums_hbm, counts_hbm, idx_v, ones_v, zc_v,
             st_v, acc2_sh, cnt_sh, sem, sem_i, sem_s):
    c = lax.axis_index("c")
    s = lax.axis_index("s")
    wid = c * NS + s

    zero16 = jnp.zeros((LANES,), jnp.float32)
    ones16 = jnp.ones((LANES,), jnp.float32)

    for k in range(8):
        ones_v[pl.ds(k * LANES, LANES)] = ones16

    def zc(i, _):
        zc_v[pl.ds(i * LANES, LANES)] = zero16
        return 0
    lax.fori_loop(0, 40, zc, 0)

    for d in range(D_EDGE):
        pltpu.sync_copy(zc_v, acc2_sh.at[d, pl.ds(s * 640, 640)])
    pltpu.sync_copy(zc_v, cnt_sh.at[pl.ds(s * 640, 640)])

    plsc.subcore_barrier()

    start = ROWS_BASE * wid + jnp.minimum(wid, ROWS_REM)
    cnt = ROWS_BASE + jnp.where(wid < ROWS_REM, 1, 0)
    end = start + cnt

    def issue_loads(r, slot):
        pltpu.async_copy(tgt_hbm.at[0, pl.ds(r * 128, 128)], idx_v.at[slot],
                         sem_i)
        pltpu.async_copy(et_hbm.at[:, pl.ds(r * 128, 128)], st_v.at[slot],
                         sem_s)

    issue_loads(start, 0)

    def body(r, _):
        slot = lax.rem(r - start, 2)
        pltpu.make_async_copy(tgt_hbm.at[0, pl.ds(r * 128, 128)],
                              idx_v.at[slot], sem_i).wait()
        pltpu.make_async_copy(et_hbm.at[:, pl.ds(r * 128, 128)],
                              st_v.at[slot], sem_s).wait()

        @pl.when(r + 1 < end)
        def _():
            issue_loads(r + 1, 1 - slot)

        hs = [
            pltpu.async_copy(st_v.at[slot, d], acc2_sh.at[d].at[idx_v.at[slot]],
                             sem, add=True)
            for d in range(D_EDGE)
        ]
        hc = pltpu.async_copy(ones_v, cnt_sh.at[idx_v.at[slot]], sem, add=True)
        for h in hs:
            h.wait()
        hc.wait()
        return 0

    lax.fori_loop(start, end, body, 0)

    plsc.subcore_barrier()

    pltpu.sync_copy(acc2_sh.at[:, pl.ds(s * 640, 640)],
                    sums_hbm.at[c, :, pl.ds(s * 640, 640)])
    pltpu.sync_copy(cnt_sh.at[pl.ds(s * 640, 640)],
                    counts_hbm.at[c, pl.ds(s * 640, 640)])


def _sc_segment_sum(edge_attr_t, edge_index):
    mesh = plsc.VectorSubcoreMesh(
        core_axis_name="c", subcore_axis_name="s", num_cores=NC,
        num_subcores=NS)
    f = functools.partial(
        pl.kernel,
        out_type=[
            jax.ShapeDtypeStruct((NC, D_EDGE, NPAD), jnp.float32),
            jax.ShapeDtypeStruct((NC, NPAD), jnp.float32),
        ],
        mesh=mesh,
        compiler_params=pltpu.CompilerParams(
            needs_layout_passes=False, use_tc_tiling_on_sc=False),
        scratch_types=[
            pltpu.VMEM((2, 128), jnp.int32),
            pltpu.VMEM((128,), jnp.float32),
            pltpu.VMEM((640,), jnp.float32),
            pltpu.VMEM((2, D_EDGE, 128), jnp.float32),
            pltpu.VMEM_SHARED((D_EDGE, NPAD), jnp.float32),
            pltpu.VMEM_SHARED((NPAD,), jnp.float32),
            pltpu.SemaphoreType.DMA,
            pltpu.SemaphoreType.DMA,
            pltpu.SemaphoreType.DMA,
        ],
    )(_sc_body)
    return f(edge_attr_t, edge_index)


def _tc_body(node_ref, sums_ref, counts_ref, wn_ref, we_ref, b_ref, out_ref):
    s_t = sums_ref[0] + sums_ref[1]
    cnts = counts_ref[0] + counts_ref[1]
    mean_t = s_t / jnp.maximum(cnts, 1.0)[None, :]
    acc = jnp.dot(node_ref[...], wn_ref[...], preferred_element_type=jnp.float32)
    acc += lax.dot_general(mean_t, we_ref[...], (((0,), (0,)), ((), ())),
                           preferred_element_type=jnp.float32)
    out_ref[...] = jax.nn.sigmoid(acc + b_ref[...])


def _tc_finish(node_attr, sums, counts, wn, we, b2d):
    blk = 1024
    grid = pl.cdiv(N_NODES, blk)
    return pl.pallas_call(
        _tc_body,
        grid=(grid,),
        in_specs=[
            pl.BlockSpec((blk, D_IN), lambda i: (i, 0)),
            pl.BlockSpec((NC, D_EDGE, blk), lambda i: (0, 0, i)),
            pl.BlockSpec((NC, blk), lambda i: (0, i)),
            pl.BlockSpec((D_IN, D_OUT), lambda i: (0, 0)),
            pl.BlockSpec((D_EDGE, D_OUT), lambda i: (0, 0)),
            pl.BlockSpec((1, D_OUT), lambda i: (0, 0)),
        ],
        out_specs=pl.BlockSpec((blk, D_OUT), lambda i: (i, 0)),
        out_shape=jax.ShapeDtypeStruct((N_NODES, D_OUT), jnp.float32),
    )(node_attr, sums, counts, wn, we, b2d)


@jax.jit
def kernel(edge_attr, edge_index, node_attr, W, b):
    sums, counts = _sc_segment_sum(edge_attr.T, edge_index)
    wn = W[:, :D_IN].T
    we = W[:, D_IN:].T
    return _tc_finish(node_attr, sums, counts, wn, we, b.reshape(1, D_OUT))
